# baseline (speedup 1.0000x reference)
import functools
import numpy as np
import jax
import jax.numpy as jnp
from jax import lax
from jax.experimental import pallas as pl
from jax.experimental.pallas import tpu as pltpu
from jax.experimental.pallas import tpu_sc as plsc

_B, _L = 16384, 200
_LOGEPS = np.float32(np.log(np.float32(1e-10)))
_NW = 32
_ROWS_PER_W = _B // _NW
_RCHUNK = 64
_NCHUNK = _ROWS_PER_W // _RCHUNK
_CL = _RCHUNK * _L
_NVEC = (_L + 15) // 16
_TAIL = _L - (_NVEC - 1) * 16
_NG = (_L + 63) // 64
_LASTW = (_L - (_NG - 1) * 64) // 4

_mesh = plsc.VectorSubcoreMesh(core_axis_name="c", subcore_axis_name="s")


@functools.partial(
    pl.kernel,
    out_type=jax.ShapeDtypeStruct((_B, _L), jnp.float32),
    mesh=_mesh,
    scratch_types=[
        pltpu.VMEM((_CL + 16,), jnp.float32),
        pltpu.VMEM((_CL + 16,), jnp.float32),
        pltpu.VMEM((_RCHUNK, _L), jnp.uint8),
        pltpu.VMEM((_RCHUNK, _L), jnp.uint8),
        pltpu.VMEM((_RCHUNK, _L), jnp.float32),
        pltpu.VMEM((_RCHUNK, _L), jnp.float32),
        pltpu.VMEM((208,), jnp.float32),
        pltpu.SemaphoreType.DMA,
        pltpu.SemaphoreType.DMA,
        pltpu.SemaphoreType.DMA,
        pltpu.SemaphoreType.DMA,
        pltpu.SemaphoreType.DMA,
        pltpu.SemaphoreType.DMA,
    ],
    compiler_params=pltpu.CompilerParams(
        needs_layout_passes=False, use_tc_tiling_on_sc=False),
)
def _restore(logits_hbm, maskb_hbm, out_hbm,
             lbuf0, lbuf1, mbuf0, mbuf1, obuf0, obuf1, bsum,
             sl0, sl1, sm0, sm1, so0, so1):
    wid = lax.axis_index("s") * 2 + lax.axis_index("c")
    iota = lax.iota(jnp.int32, 16)
    lane15 = jnp.full((16,), 15, jnp.int32)
    zero16 = jnp.zeros((16,), jnp.float32)
    lbuf = (lbuf0, lbuf1)
    mbuf = (mbuf0, mbuf1)
    obuf = (obuf0, obuf1)
    sl = (sl0, sl1)
    sm = (sm0, sm1)
    so = (so0, so1)

    def start_in(c):
        p = c & 1
        row0 = wid * _ROWS_PER_W + c * _RCHUNK
        hl = pltpu.async_copy(
            logits_hbm.at[pl.ds(row0 * _L, _CL)], lbuf[p].at[pl.ds(0, _CL)], sl[p])
        hm = pltpu.async_copy(
            maskb_hbm.at[pl.ds(row0, _RCHUNK), :], mbuf[p], sm[p])
        return hl, hm

    def make_row_body(p):
        lb, mb, ob = lbuf[p], mbuf[p], obuf[p]

        def row_body(r, _):
            ro = r * _L
            rsplat = jnp.zeros((16,), jnp.int32) + r
            ws = []
            wexc_bs = []
            wexcls = []
            carry = jnp.zeros((16,), jnp.int32)
            for g in range(_NG):
                if g == _NG - 1:
                    raw = mb[r, pl.ds(_L - 64, 64)]
                    wv = plsc.bitcast(raw, jnp.int32)
                    w = wv.at[jnp.minimum(iota + (16 - _LASTW), lane15)].get(
                        mode="promise_in_bounds")
                    w = jnp.where(iota < _LASTW, w, 0)
                else:
                    raw = mb[r, pl.ds(g * 64, 64)]
                    w = plsc.bitcast(raw, jnp.int32)
                winc = w * jnp.int32(0x01010101)
                wordtot = lax.shift_right_logical(winc, jnp.full((16,), 24, jnp.int32))
                lanecum = plsc.cumsum(wordtot)
                ws.append(w)
                wexc_bs.append(winc << 8)
                wexcls.append(lanecum - wordtot + carry)
                carry = carry + lanecum.at[lane15].get(mode="promise_in_bounds")
            nv_vec = carry
            nv_safe = jnp.maximum(nv_vec, 1)
            is_fast = jnp.any(nv_vec >= 16)
            has_valid = jnp.any(nv_vec > 0)

            for kv in range(_NVEC):
                bsum[pl.ds(kv * 16, 16)] = zero16

            @pl.when(is_fast)
            def _():
                idx = iota
                for jv in range(_NVEC):
                    v = lb[pl.ds(ro + jv * 16, 16)]
                    if jv == _NVEC - 1:
                        v = jnp.where(iota < _TAIL, v, 0.0)
                    plsc.addupdate_scatter(bsum, [idx], v)
                    if jv != _NVEC - 1:
                        nxt = idx + 16
                        idx = jnp.where(nxt >= nv_vec, nxt - nv_vec, nxt)

            @pl.when(jnp.logical_not(is_fast))
            def _():
                nv_s = jnp.maximum(jnp.max(nv_vec), 1)
                qmax = jnp.int32(_L - 1) // nv_s + 1

                def qstep(q, acc):
                    off = q * nv_s
                    v = lb[pl.ds(ro + off, 16)]
                    return acc + jnp.where(iota + off < _L, v, 0.0)

                bsum[pl.ds(0, 16)] = lax.fori_loop(0, qmax, qstep, zero16)

            nv_f = nv_safe.astype(jnp.float32)
            q_vec = (jnp.float32(_L) / nv_f).astype(jnp.int32)
            r_vec = _L - q_vec * nv_safe
            inv_hi = 1.0 / (q_vec + 1).astype(jnp.float32)
            inv_lo = 1.0 / q_vec.astype(jnp.float32)

            for g in range(_NG):
                w, wexc_b, wexcl = ws[g], wexc_bs[g], wexcls[g]
                for b in range(4):
                    m = (w >> (8 * b)) & 1
                    rank = wexcl + ((wexc_b >> (8 * b)) & 0xFF)
                    gval = plsc.load_gather(bsum, [rank])
                    inv = jnp.where(rank < r_vec, inv_hi, inv_lo)
                    val = jnp.where(m > 0, gval * inv, _LOGEPS)
                    col = g * 64 + iota * 4 + b
                    if g == _NG - 1:
                        plsc.store_scatter(ob, [rsplat, col], val, mask=col < _L)
                    else:
                        plsc.store_scatter(ob, [rsplat, col], val)

            @pl.when(jnp.logical_not(has_valid))
            def _():
                mv = bsum[pl.ds(0, 16)] * inv_lo
                plsc.store_scatter(ob, [rsplat, iota], mv, mask=iota < 1)

            return 0

        return row_body

    in_handles = start_in(0)
    out_handles = [None, None]
    for c in range(_NCHUNK):
        p = c & 1
        for h in in_handles:
            h.wait()
        if c + 1 < _NCHUNK:
            in_handles = start_in(c + 1)
        if out_handles[p] is not None:
            out_handles[p].wait()
        lax.fori_loop(0, _RCHUNK, make_row_body(p), 0)
        row0 = wid * _ROWS_PER_W + c * _RCHUNK
        out_handles[p] = pltpu.async_copy(
            obuf[p], out_hbm.at[pl.ds(row0, _RCHUNK), :], so[p])
    for h in out_handles:
        if h is not None:
            h.wait()


def kernel(flattened_logits, mask):
    return _restore(flattened_logits, mask.view(jnp.uint8))


# speedup vs baseline: 1.0574x; 1.0574x over previous
import functools
import numpy as np
import jax
import jax.numpy as jnp
from jax import lax
from jax.experimental import pallas as pl
from jax.experimental.pallas import tpu as pltpu
from jax.experimental.pallas import tpu_sc as plsc

_B, _L = 16384, 200
_LOGEPS = np.float32(np.log(np.float32(1e-10)))
_NW = 32
_ROWS_PER_W = _B // _NW
_RCHUNK = 64
_NCHUNK = _ROWS_PER_W // _RCHUNK
_CL = _RCHUNK * _L
_NVEC = (_L + 15) // 16
_TAIL = _L - (_NVEC - 1) * 16
_NG = (_L + 63) // 64
_LASTW = (_L - (_NG - 1) * 64) // 4

_mesh = plsc.VectorSubcoreMesh(core_axis_name="c", subcore_axis_name="s")


@functools.partial(
    pl.kernel,
    out_type=jax.ShapeDtypeStruct((_B, _L), jnp.float32),
    mesh=_mesh,
    scratch_types=[
        pltpu.VMEM((_CL + 16,), jnp.float32),
        pltpu.VMEM((_CL + 16,), jnp.float32),
        pltpu.VMEM((_CL + 64,), jnp.uint8),
        pltpu.VMEM((_CL + 64,), jnp.uint8),
        pltpu.VMEM((_RCHUNK, _L), jnp.float32),
        pltpu.VMEM((_RCHUNK, _L), jnp.float32),
        pltpu.VMEM((208,), jnp.float32),
        pltpu.VMEM((208,), jnp.float32),
        pltpu.SemaphoreType.DMA,
        pltpu.SemaphoreType.DMA,
        pltpu.SemaphoreType.DMA,
        pltpu.SemaphoreType.DMA,
        pltpu.SemaphoreType.DMA,
        pltpu.SemaphoreType.DMA,
    ],
    compiler_params=pltpu.CompilerParams(
        needs_layout_passes=False, use_tc_tiling_on_sc=False),
)
def _restore(logits_hbm, maskb_hbm, out_hbm,
             lbuf0, lbuf1, mbuf0, mbuf1, obuf0, obuf1, bsum_a, bsum_b,
             sl0, sl1, sm0, sm1, so0, so1):
    wid = lax.axis_index("s") * 2 + lax.axis_index("c")
    iota = lax.iota(jnp.int32, 16)
    lane15 = jnp.full((16,), 15, jnp.int32)
    zero16 = jnp.zeros((16,), jnp.float32)
    lbuf = (lbuf0, lbuf1)
    mbuf = (mbuf0, mbuf1)
    obuf = (obuf0, obuf1)
    sl = (sl0, sl1)
    sm = (sm0, sm1)
    so = (so0, so1)

    def start_in(c):
        p = c & 1
        row0 = wid * _ROWS_PER_W + c * _RCHUNK
        hl = pltpu.async_copy(
            logits_hbm.at[pl.ds(row0 * _L, _CL)], lbuf[p].at[pl.ds(0, _CL)], sl[p])
        hm = pltpu.async_copy(
            maskb_hbm.at[pl.ds(row0 * _L, _CL)], mbuf[p].at[pl.ds(0, _CL)], sm[p])
        return hl, hm

    def do_row(lb, mb, ob, bsum, r):
        ro = r * _L
        rsplat = jnp.zeros((16,), jnp.int32) + r
        ws = []
        wexc_bs = []
        wexcls = []
        carry = jnp.zeros((16,), jnp.int32)
        for g in range(_NG):
            raw = mb[pl.ds(ro + g * 64, 64)]
            w = plsc.bitcast(raw, jnp.int32)
            if g == _NG - 1:
                w = jnp.where(iota < _LASTW, w, 0)
            winc = w * jnp.int32(0x01010101)
            wordtot = lax.shift_right_logical(winc, jnp.full((16,), 24, jnp.int32))
            lanecum = plsc.cumsum(wordtot)
            ws.append(w)
            wexc_bs.append(winc << 8)
            wexcls.append(lanecum - wordtot + carry)
            carry = carry + lanecum.at[lane15].get(mode="promise_in_bounds")
        nv_vec = carry
        nv_safe = jnp.maximum(nv_vec, 1)
        is_fast_vec = nv_vec >= 16
        is_slow = jnp.any(nv_vec < 16)
        has_valid = jnp.any(nv_vec > 0)

        for kv in range(_NVEC):
            bsum[pl.ds(kv * 16, 16)] = zero16

        idx = iota
        for jv in range(_NVEC):
            v = lb[pl.ds(ro + jv * 16, 16)]
            if jv == _NVEC - 1:
                v = jnp.where(iota < _TAIL, v, 0.0)
            safe_idx = jnp.where(is_fast_vec, idx, iota + jv * 16)
            plsc.addupdate_scatter(bsum, [safe_idx], v)
            if jv != _NVEC - 1:
                nxt = idx + 16
                idx = jnp.where(nxt >= nv_vec, nxt - nv_vec, nxt)

        @pl.when(is_slow)
        def _():
            nv_s = jnp.maximum(jnp.max(nv_vec), 1)
            qmax = jnp.int32(_L - 1) // nv_s + 1

            def qstep(q, acc):
                off = q * nv_s
                v = lb[pl.ds(ro + off, 16)]
                return acc + jnp.where(iota + off < _L, v, 0.0)

            bsum[pl.ds(0, 16)] = lax.fori_loop(0, qmax, qstep, zero16)

        nv_f = nv_safe.astype(jnp.float32)
        q_vec = (jnp.float32(_L) / nv_f).astype(jnp.int32)
        r_vec = _L - q_vec * nv_safe
        inv_hi = 1.0 / (q_vec + 1).astype(jnp.float32)
        inv_lo = 1.0 / q_vec.astype(jnp.float32)

        iota4 = iota * 4
        for g in range(_NG):
            w, wexc_b, wexcl = ws[g], wexc_bs[g], wexcls[g]
            for b in range(4):
                m = (w >> (8 * b)) & 1
                rank = wexcl + ((wexc_b >> (8 * b)) & 0xFF)
                gval = plsc.load_gather(bsum, [rank])
                inv = jnp.where(rank < r_vec, inv_hi, inv_lo)
                val = jnp.where(m > 0, gval * inv, _LOGEPS)
                col = iota4 + (g * 64 + b)
                if g == _NG - 1:
                    plsc.store_scatter(ob, [rsplat, col], val, mask=col < _L)
                else:
                    plsc.store_scatter(ob, [rsplat, col], val)

        @pl.when(jnp.logical_not(has_valid))
        def _():
            mv = bsum[pl.ds(0, 16)] * inv_lo
            plsc.store_scatter(ob, [rsplat, iota], mv, mask=iota < 1)

    def make_pair_body(p):
        lb, mb, ob = lbuf[p], mbuf[p], obuf[p]

        def pair_body(i, _):
            do_row(lb, mb, ob, bsum_a, i * 2)
            do_row(lb, mb, ob, bsum_b, i * 2 + 1)
            return 0

        return pair_body

    in_handles = start_in(0)
    out_handles = [None, None]
    for c in range(_NCHUNK):
        p = c & 1
        for h in in_handles:
            h.wait()
        if c + 1 < _NCHUNK:
            in_handles = start_in(c + 1)
        if out_handles[p] is not None:
            out_handles[p].wait()
        lax.fori_loop(0, _RCHUNK // 2, make_pair_body(p), 0)
        row0 = wid * _ROWS_PER_W + c * _RCHUNK
        out_handles[p] = pltpu.async_copy(
            obuf[p], out_hbm.at[pl.ds(row0, _RCHUNK), :], so[p])
    for h in out_handles:
        if h is not None:
            h.wait()


def kernel(flattened_logits, mask):
    return _restore(flattened_logits, mask.view(jnp.uint8).reshape(_B * _L))
